# Initial kernel scaffold; baseline (speedup 1.0000x reference)
import jax
import jax.numpy as jnp
from jax.experimental import pallas as pl


def kernel(input_tensor):
    raise NotImplementedError("write your pallas kernel here")


# speedup vs baseline: 3.2383x; 3.2383x over previous
import jax
import jax.numpy as jnp
from jax.experimental import pallas as pl
from jax.experimental.pallas import tpu as pltpu

_N_ITERS = 21
_INV_TEMPERATURE = 25.0
_TINY = 1e-30
_BLOCK_B = 8


def _sinkhorn_block(x_ref, o_ref):
    la = x_ref[...] * _INV_TEMPERATURE
    m = jnp.max(la, axis=2, keepdims=True)
    p = jnp.exp(la - m)

    def body(_, p):
        rs = jnp.sum(p, axis=2, keepdims=True)
        p = p * (1.0 / jnp.maximum(rs, _TINY))
        cs = jnp.sum(p, axis=1, keepdims=True)
        p = p * (1.0 / jnp.maximum(cs, _TINY))
        return p

    o_ref[...] = jax.lax.fori_loop(0, _N_ITERS, body, p)


def kernel(input_tensor):
    b, n, _ = input_tensor.shape
    grid = (b // _BLOCK_B,)
    return pl.pallas_call(
        _sinkhorn_block,
        out_shape=jax.ShapeDtypeStruct(input_tensor.shape, input_tensor.dtype),
        grid=grid,
        in_specs=[pl.BlockSpec((_BLOCK_B, n, n), lambda i: (i, 0, 0))],
        out_specs=pl.BlockSpec((_BLOCK_B, n, n), lambda i: (i, 0, 0)),
        compiler_params=pltpu.CompilerParams(
            dimension_semantics=("parallel",),
        ),
        name="sinkhorn",
    )(input_tensor)


# speedup vs baseline: 4.0786x; 1.2595x over previous
import jax
import jax.numpy as jnp
from jax.experimental import pallas as pl
from jax.experimental.pallas import tpu as pltpu

_N_ITERS = 21
_INV_TEMPERATURE = 25.0
_TINY = 1e-30
_BLOCK_B = 8


def _sinkhorn_block(x_ref, o_ref):
    for mm in range(_BLOCK_B):
        la = x_ref[mm] * _INV_TEMPERATURE
        m = jnp.max(la, axis=1, keepdims=True)
        o_ref[mm] = jnp.exp(la - m)

    def half_steps(c):
        rs, prods = [], []
        for mm in range(_BLOCK_B):
            k = o_ref[mm]
            u = jnp.sum(k * c[mm], axis=1, keepdims=True)
            r = 1.0 / jnp.maximum(u, _TINY)
            rs.append(r)
            prods.append(k * r)
        return rs, prods

    def body(_, c):
        _, prods = half_steps(c)
        news = []
        for mm in range(_BLOCK_B):
            v = jnp.sum(prods[mm], axis=0)
            news.append(1.0 / jnp.maximum(v, _TINY))
        return jnp.stack(news)

    c0 = jnp.ones((_BLOCK_B, 128), jnp.float32)
    c = jax.lax.fori_loop(0, _N_ITERS - 1, body, c0)

    _, prods = half_steps(c)
    for mm in range(_BLOCK_B):
        v = jnp.sum(prods[mm], axis=0, keepdims=True)
        cf = 1.0 / jnp.maximum(v, _TINY)
        o_ref[mm] = prods[mm] * cf


def kernel(input_tensor):
    b, n, _ = input_tensor.shape
    grid = (b // _BLOCK_B,)
    return pl.pallas_call(
        _sinkhorn_block,
        out_shape=jax.ShapeDtypeStruct(input_tensor.shape, input_tensor.dtype),
        grid=grid,
        in_specs=[pl.BlockSpec((_BLOCK_B, n, n), lambda i: (i, 0, 0))],
        out_specs=pl.BlockSpec((_BLOCK_B, n, n), lambda i: (i, 0, 0)),
        compiler_params=pltpu.CompilerParams(
            dimension_semantics=("parallel",),
        ),
        name="sinkhorn",
    )(input_tensor)


# speedup vs baseline: 5.8025x; 1.4227x over previous
import jax
import jax.numpy as jnp
from jax.experimental import pallas as pl
from jax.experimental.pallas import tpu as pltpu

_N_ITERS = 21
_INV_TEMPERATURE = 25.0
_TINY = 1e-30
_BLOCK_B = 8


def _sinkhorn_block(x_ref, o_ref):
    for mm in range(_BLOCK_B):
        la = x_ref[mm] * _INV_TEMPERATURE
        m = jnp.max(la, axis=1, keepdims=True)
        o_ref[mm] = jnp.exp(la - m)

    def half_steps(c):
        rs, prods = [], []
        for mm in range(_BLOCK_B):
            k = o_ref[mm]
            u = jnp.sum(k * c[mm], axis=1, keepdims=True)
            r = 1.0 / jnp.maximum(u, _TINY)
            rs.append(r)
            prods.append(k * r)
        return rs, prods

    def body(_, c):
        _, prods = half_steps(c)
        news = []
        for mm in range(_BLOCK_B):
            v = jnp.sum(prods[mm], axis=0)
            news.append(1.0 / jnp.maximum(v, _TINY))
        return jnp.stack(news)

    c0 = jnp.ones((_BLOCK_B, 128), jnp.float32)
    c = jax.lax.fori_loop(0, _N_ITERS - 1, body, c0)

    _, prods = half_steps(c)
    for mm in range(_BLOCK_B):
        v = jnp.sum(prods[mm], axis=0, keepdims=True)
        cf = 1.0 / jnp.maximum(v, _TINY)
        o_ref[mm] = prods[mm] * cf


def _sinkhorn_pallas(x):
    b, n, _ = x.shape
    grid = (b // _BLOCK_B,)
    return pl.pallas_call(
        _sinkhorn_block,
        out_shape=jax.ShapeDtypeStruct(x.shape, x.dtype),
        grid=grid,
        in_specs=[pl.BlockSpec((_BLOCK_B, n, n), lambda i: (i, 0, 0))],
        out_specs=pl.BlockSpec((_BLOCK_B, n, n), lambda i: (i, 0, 0)),
        compiler_params=pltpu.CompilerParams(
            dimension_semantics=("parallel",),
        ),
        name="sinkhorn",
    )(x)


def kernel(input_tensor):
    devs = jax.devices()
    b = input_tensor.shape[0]
    nd = len(devs)
    while nd > 1 and b % (nd * _BLOCK_B) != 0:
        nd -= 1
    if nd <= 1:
        return _sinkhorn_pallas(input_tensor)
    mesh = jax.sharding.Mesh(devs[:nd], ("b",))
    pspec = jax.sharding.PartitionSpec("b")
    fn = jax.shard_map(
        _sinkhorn_pallas, mesh=mesh, in_specs=pspec, out_specs=pspec,
        check_vma=False,
    )
    x = jax.lax.with_sharding_constraint(
        input_tensor, jax.sharding.NamedSharding(mesh, pspec))
    return fn(x)
